# baseline (speedup 1.0000x reference)
import functools

import jax
import jax.numpy as jnp
from jax import lax
from jax.experimental import pallas as pl
from jax.experimental.pallas import tpu as pltpu
from jax.experimental.pallas import tpu_sc as plsc

T = 16384
E = 256
K = 8
NGROUP = 8
GSIZE = E // NGROUP
NC, NS, L = 2, 16, 16
NW = NC * NS
TPW = T // NW
CHUNK = 128
NCHUNK = TPW // CHUNK
NEG_INF = float("-inf")


def _merge_top16(ka, va, kb, vb):
    kr = lax.rev(kb, (0,))
    vr = lax.rev(vb, (0,))
    ge = ka >= kr
    hk = jnp.where(ge, ka, kr)
    hv = jnp.where(ge, va, vr)
    return plsc.sort_key_val(hk, hv, descending=True)


def _routing_body(lf_hbm, bias_hbm, oi_hbm, ov_hbm,
                  lbuf, bias_v, hk_scr, hid_scr, lo_scr,
                  oi_scr, ov_scr):
    wid = lax.axis_index("s") * NC + lax.axis_index("c")
    iota = lax.iota(jnp.int32, L)
    pltpu.sync_copy(bias_hbm, bias_v)
    tok0 = wid * TPW

    @pl.loop(0, NCHUNK)
    def _chunk(ci):
        base_tok = tok0 + ci * CHUNK
        pltpu.sync_copy(lf_hbm.at[pl.ds(base_tok * E, CHUNK * E)], lbuf)

        @pl.loop(0, CHUNK)
        def _tok(t):
            toff = t * E
            gv = jnp.full((L,), NEG_INF, jnp.float32)
            for g in range(NGROUP):
                xa = lbuf[pl.ds(toff + g * GSIZE, L)]
                xb = lbuf[pl.ds(toff + g * GSIZE + L, L)]
                sa = 1.0 / (1.0 + jnp.exp(-xa))
                sb = 1.0 / (1.0 + jnp.exp(-xb))
                a = sa + bias_v[pl.ds(g * GSIZE, L)]
                b = sb + bias_v[pl.ds(g * GSIZE + L, L)]
                ge = a >= b
                hi = jnp.where(ge, a, b)
                lo = jnp.where(ge, b, a)
                hi_src = jnp.where(ge, g * GSIZE + iota, g * GSIZE + L + iota)
                hk, hid = plsc.sort_key_val(hi, hi_src, descending=True)
                hk_scr[pl.ds(g * L, L)] = hk
                hid_scr[pl.ds(g * L, L)] = hid
                lo_scr[pl.ds(g * L, L)] = lo
                psm = jnp.max(a + b)
                s = jnp.maximum(hk[0] + hk[1], psm)
                gv = jnp.where(iota == g, s, gv)
            _, gid = plsc.sort_key_val(gv, iota, descending=True)
            kept = []
            for r in range(4):
                base = gid[r] * L
                hk_r = hk_scr[pl.ds(base, L)]
                hid_r = hid_scr[pl.ds(base, L)]
                hid_sh = plsc.load_gather(hid_scr, [base + ((iota - 8) & (L - 1))])
                glo_sh = plsc.load_gather(lo_scr, [base + (hid_sh & (L - 1))])
                mlow = iota < 8
                cw = jnp.where(mlow, hk_r, glo_sh)
                cid = jnp.where(mlow, hid_r, hid_sh ^ L)
                kept.append(plsc.sort_key_val(cw, cid, descending=True))
            u0 = _merge_top16(*kept[0], *kept[1])
            u1 = _merge_top16(*kept[2], *kept[3])
            fk, fv = _merge_top16(*u0, *u1)
            mask8 = iota < K
            lg = plsc.load_gather(lbuf, [fv + toff])
            s16 = 1.0 / (1.0 + jnp.exp(-lg))
            s8 = jnp.where(mask8, s16, 0.0)
            denom = jnp.broadcast_to(jnp.sum(s8) + 1e-20, (L,))
            vals = s8 * 2.5 / denom
            keys = jnp.where(mask8, vals, -1.0)
            ok, oi = plsc.sort_key_val(keys, fv, descending=True)
            plsc.store_compressed(ov_scr.at[pl.ds(t * K, L)], ok, mask=mask8)
            plsc.store_compressed(oi_scr.at[pl.ds(t * K, L)], oi, mask=mask8)

        pltpu.sync_copy(ov_scr.at[pl.ds(0, CHUNK * K)],
                        ov_hbm.at[pl.ds(base_tok * K, CHUNK * K)])
        pltpu.sync_copy(oi_scr.at[pl.ds(0, CHUNK * K)],
                        oi_hbm.at[pl.ds(base_tok * K, CHUNK * K)])


_router = functools.partial(
    pl.kernel,
    out_type=(
        jax.ShapeDtypeStruct((T * K,), jnp.int32),
        jax.ShapeDtypeStruct((T * K,), jnp.float32),
    ),
    mesh=plsc.VectorSubcoreMesh(
        core_axis_name="c", subcore_axis_name="s", num_cores=NC, num_subcores=NS
    ),
    compiler_params=pltpu.CompilerParams(needs_layout_passes=False),
    scratch_types=[
        pltpu.VMEM((CHUNK * E,), jnp.float32),
        pltpu.VMEM((E,), jnp.float32),
        pltpu.VMEM((NGROUP * L,), jnp.float32),
        pltpu.VMEM((NGROUP * L,), jnp.int32),
        pltpu.VMEM((NGROUP * L,), jnp.float32),
        pltpu.VMEM((CHUNK * K + K,), jnp.int32),
        pltpu.VMEM((CHUNK * K + K,), jnp.float32),
    ],
)(_routing_body)


def kernel(logits, e_score_correction_bias):
    oi, ov = _router(logits.reshape(-1), e_score_correction_bias)
    return oi.reshape(T, K), ov.reshape(T, K)


# speedup vs baseline: 1.0293x; 1.0293x over previous
import functools

import jax
import jax.numpy as jnp
from jax import lax
from jax.experimental import pallas as pl
from jax.experimental.pallas import tpu as pltpu
from jax.experimental.pallas import tpu_sc as plsc

T = 16384
E = 256
K = 8
NGROUP = 8
GSIZE = E // NGROUP
NC, NS, L = 2, 16, 16
NW = NC * NS
TPW = T // NW
CHUNK = 128
NCHUNK = TPW // CHUNK
NEG_INF = float("-inf")


def _merge_top16(ka, va, kb, vb):
    kr = lax.rev(kb, (0,))
    vr = lax.rev(vb, (0,))
    ge = ka >= kr
    hk = jnp.where(ge, ka, kr)
    hv = jnp.where(ge, va, vr)
    return plsc.sort_key_val(hk, hv, descending=True)


def _routing_body(lf_hbm, bias_hbm, oi_hbm, ov_hbm,
                  lbuf, bias_v, sc_scr, hk_scr, hid_scr, lo_scr,
                  oi_scr, ov_scr):
    wid = lax.axis_index("s") * NC + lax.axis_index("c")
    iota = lax.iota(jnp.int32, L)
    pltpu.sync_copy(bias_hbm, bias_v)
    tok0 = wid * TPW

    @pl.loop(0, NCHUNK)
    def _chunk(ci):
        base_tok = tok0 + ci * CHUNK
        pltpu.sync_copy(lf_hbm.at[pl.ds(base_tok * E, CHUNK * E)], lbuf)

        @pl.loop(0, CHUNK)
        def _tok(t):
            toff = t * E
            gv = jnp.full((L,), NEG_INF, jnp.float32)
            for g in range(NGROUP):
                xa = lbuf[pl.ds(toff + g * GSIZE, L)]
                xb = lbuf[pl.ds(toff + g * GSIZE + L, L)]
                sa = 1.0 / (1.0 + jnp.exp(-xa))
                sb = 1.0 / (1.0 + jnp.exp(-xb))
                sc_scr[pl.ds(g * GSIZE, L)] = sa
                sc_scr[pl.ds(g * GSIZE + L, L)] = sb
                a = sa + bias_v[pl.ds(g * GSIZE, L)]
                b = sb + bias_v[pl.ds(g * GSIZE + L, L)]
                ge = a >= b
                hi = jnp.where(ge, a, b)
                lo = jnp.where(ge, b, a)
                hi_src = jnp.where(ge, g * GSIZE + iota, g * GSIZE + L + iota)
                hk, hid = plsc.sort_key_val(hi, hi_src, descending=True)
                hk_scr[pl.ds(g * L, L)] = hk
                hid_scr[pl.ds(g * L, L)] = hid
                lo_scr[pl.ds(g * L, L)] = lo
                psm = jnp.max(a + b)
                s = jnp.maximum(hk[0] + hk[1], psm)
                gv = jnp.where(iota == g, s, gv)
            _, gid = plsc.sort_key_val(gv, iota, descending=True)
            kept = []
            for r in range(4):
                base = gid[r] * L
                hk_r = hk_scr[pl.ds(base, L)]
                hid_r = hid_scr[pl.ds(base, L)]
                hid_sh = plsc.load_gather(hid_scr, [base + ((iota - 8) & (L - 1))])
                glo_sh = plsc.load_gather(lo_scr, [base + (hid_sh & (L - 1))])
                mlow = iota < 8
                cw = jnp.where(mlow, hk_r, glo_sh)
                cid = jnp.where(mlow, hid_r, hid_sh ^ L)
                kept.append(plsc.sort_key_val(cw, cid, descending=True))
            u0 = _merge_top16(*kept[0], *kept[1])
            u1 = _merge_top16(*kept[2], *kept[3])
            fk, fv = _merge_top16(*u0, *u1)
            mask8 = iota < K
            sgath = plsc.load_gather(sc_scr, [fv])
            s8 = jnp.where(mask8, sgath, 0.0)
            denom = jnp.broadcast_to(jnp.sum(s8) + 1e-20, (L,))
            vals = s8 * 2.5 / denom
            keys = jnp.where(mask8, vals, -1.0)
            ok, oi = plsc.sort_key_val(keys, fv, descending=True)
            plsc.store_compressed(ov_scr.at[pl.ds(t * K, L)], ok, mask=mask8)
            plsc.store_compressed(oi_scr.at[pl.ds(t * K, L)], oi, mask=mask8)

        pltpu.sync_copy(ov_scr.at[pl.ds(0, CHUNK * K)],
                        ov_hbm.at[pl.ds(base_tok * K, CHUNK * K)])
        pltpu.sync_copy(oi_scr.at[pl.ds(0, CHUNK * K)],
                        oi_hbm.at[pl.ds(base_tok * K, CHUNK * K)])


_router = functools.partial(
    pl.kernel,
    out_type=(
        jax.ShapeDtypeStruct((T * K,), jnp.int32),
        jax.ShapeDtypeStruct((T * K,), jnp.float32),
    ),
    mesh=plsc.VectorSubcoreMesh(
        core_axis_name="c", subcore_axis_name="s", num_cores=NC, num_subcores=NS
    ),
    compiler_params=pltpu.CompilerParams(needs_layout_passes=False),
    scratch_types=[
        pltpu.VMEM((CHUNK * E,), jnp.float32),
        pltpu.VMEM((E,), jnp.float32),
        pltpu.VMEM((E,), jnp.float32),
        pltpu.VMEM((NGROUP * L,), jnp.float32),
        pltpu.VMEM((NGROUP * L,), jnp.int32),
        pltpu.VMEM((NGROUP * L,), jnp.float32),
        pltpu.VMEM((CHUNK * K + K,), jnp.int32),
        pltpu.VMEM((CHUNK * K + K,), jnp.float32),
    ],
)(_routing_body)


def kernel(logits, e_score_correction_bias):
    oi, ov = _router(logits.reshape(-1), e_score_correction_bias)
    return oi.reshape(T, K), ov.reshape(T, K)
